# Initial kernel scaffold; baseline (speedup 1.0000x reference)
import jax
import jax.numpy as jnp
from jax.experimental import pallas as pl


def kernel(x, embed_weight, pe):
    raise NotImplementedError("write your pallas kernel here")


# speedup vs baseline: 2.1800x; 2.1800x over previous
import functools

import jax
import jax.numpy as jnp
from jax import lax
from jax.experimental import pallas as pl
from jax.experimental.pallas import tpu as pltpu
from jax.experimental.pallas import tpu_sc as plsc

B = 16384
P = 10
D = 32
N = B * P
NC, NS = 2, 16
NW = NC * NS
PER_W = N // NW
CH = 128
CHW = CH * D
NCHUNK = PER_W // CH
R = 14 * P
L = 16


def _sc_gather(tbl, x_flat):
    mesh = plsc.VectorSubcoreMesh(core_axis_name="c", subcore_axis_name="s")

    @functools.partial(
        pl.kernel,
        mesh=mesh,
        out_type=jax.ShapeDtypeStruct((N * D,), jnp.float32),
        scratch_types=[
            pltpu.VMEM((R * D,), jnp.float32),
            pltpu.VMEM((CH,), jnp.int32),
            pltpu.VMEM((CHW,), jnp.float32),
        ],
        compiler_params=pltpu.CompilerParams(needs_layout_passes=False),
    )
    def k(tbl_hbm, x_hbm, out_hbm, tbl_v, xbuf, rows):
        wid = lax.axis_index("s") * NC + lax.axis_index("c")
        base = wid * PER_W
        lane = lax.iota(jnp.int32, L)

        pltpu.sync_copy(tbl_hbm, tbl_v)

        def chunk(g, _):
            abs0 = base + g * CH
            pltpu.sync_copy(x_hbm.at[pl.ds(abs0, CH)], xbuf)
            for r0 in range(0, CH, L):
                pos = lax.rem(g * CH + (r0 + lane), 10)
                c16 = xbuf[pl.ds(r0, L)] * 10 + pos
                cw = c16 * D
                sconst = r0 * D + lane * D
                for j in range(D):
                    val = plsc.load_gather(tbl_v, [cw + j])
                    plsc.store_scatter(rows, [sconst + j], val)
            pltpu.sync_copy(rows, out_hbm.at[pl.ds(abs0 * D, CHW)])
            return 0

        lax.fori_loop(0, NCHUNK, chunk, 0)

    return k(tbl, x_flat)


def kernel(x, embed_weight, pe):
    tbl = (embed_weight[:, None, :] + pe[None, :, :]).reshape(R * D)
    x_flat = x.reshape(N).astype(jnp.int32)
    out = _sc_gather(tbl, x_flat)
    return out.reshape(B, P, D)


# speedup vs baseline: 2.7780x; 1.2743x over previous
import functools

import jax
import jax.numpy as jnp
from jax import lax
from jax.experimental import pallas as pl
from jax.experimental.pallas import tpu as pltpu
from jax.experimental.pallas import tpu_sc as plsc

B = 16384
P = 10
D = 32
N = B * P
NC, NS = 2, 16
NW = NC * NS
PER_W = N // NW
CH = 1280
CHW = CH * D
NCHUNK = PER_W // CH
NG = CH // 16
R = 14 * P
L = 16


def _sc_gather(tbl, x_flat):
    mesh = plsc.VectorSubcoreMesh(core_axis_name="c", subcore_axis_name="s")

    @functools.partial(
        pl.kernel,
        mesh=mesh,
        out_type=jax.ShapeDtypeStruct((N * D,), jnp.float32),
        scratch_types=[
            pltpu.VMEM((R * D,), jnp.float32),
            pltpu.VMEM((CH,), jnp.int32),
            pltpu.VMEM((CH,), jnp.int32),
            pltpu.VMEM((CHW,), jnp.float32),
            pltpu.VMEM((CHW,), jnp.float32),
            pltpu.SemaphoreType.DMA,
            pltpu.SemaphoreType.DMA,
            pltpu.SemaphoreType.DMA,
            pltpu.SemaphoreType.DMA,
        ],
        compiler_params=pltpu.CompilerParams(needs_layout_passes=False),
    )
    def k(tbl_hbm, x_hbm, out_hbm, tbl_v, xb0, xb1, rb0, rb1,
          sx0, sx1, so0, so1):
        wid = lax.axis_index("s") * NC + lax.axis_index("c")
        base = wid * PER_W
        lane = lax.iota(jnp.int32, L)
        lane32 = lane * D
        xbufs, rbufs = (xb0, xb1), (rb0, rb1)
        sxs, sos = (sx0, sx1), (so0, so1)

        pltpu.sync_copy(tbl_hbm, tbl_v)

        def load_x(g):
            return pltpu.async_copy(
                x_hbm.at[pl.ds(base + g * CH, CH)], xbufs[g % 2], sxs[g % 2])

        def gather_chunk(gc, xbuf, rows):
            def group(g2, _):
                pos = lax.rem(gc * CH + g2 * L + lane, 10)
                c16 = xbuf[pl.ds(g2 * L, L)] * 10 + pos
                cw = c16 * D
                sconst = g2 * (L * D) + lane32
                vals = [plsc.load_gather(tbl_v, [cw + j]) for j in range(D)]
                for j in range(D):
                    plsc.store_scatter(rows, [sconst + j], vals[j])
                return 0

            lax.fori_loop(0, NG, group, 0)

        x_pend = load_x(0)
        out_pend = [None, None]
        for gc in range(NCHUNK):
            b = gc % 2
            nxt = load_x(gc + 1) if gc + 1 < NCHUNK else None
            x_pend.wait()
            x_pend = nxt
            if out_pend[b] is not None:
                out_pend[b].wait()
            gather_chunk(gc, xbufs[b], rbufs[b])
            out_pend[b] = pltpu.async_copy(
                rbufs[b], out_hbm.at[pl.ds((base + gc * CH) * D, CHW)],
                sos[b])
        for h in out_pend:
            if h is not None:
                h.wait()

    return k(tbl, x_flat)


def kernel(x, embed_weight, pe):
    tbl = (embed_weight[:, None, :] + pe[None, :, :]).reshape(R * D)
    x_flat = x.reshape(N).astype(jnp.int32)
    out = _sc_gather(tbl, x_flat)
    return out.reshape(B, P, D)


# speedup vs baseline: 4.8148x; 1.7332x over previous
import functools

import jax
import jax.numpy as jnp
from jax import lax
from jax.experimental import pallas as pl
from jax.experimental.pallas import tpu as pltpu
from jax.experimental.pallas import tpu_sc as plsc

B = 16384
P = 10
D = 32
N = B * P
NC, NS = 2, 16
NW = NC * NS
PER_W = N // NW
CH = 1280
CHW = CH * D
NCHUNK = PER_W // CH
NG = CH // 16
R = 14 * P
L = 16


def _sc_gather(tbl, x_flat):
    mesh = plsc.VectorSubcoreMesh(core_axis_name="c", subcore_axis_name="s")

    @functools.partial(
        pl.kernel,
        mesh=mesh,
        out_type=jax.ShapeDtypeStruct((N * D,), jnp.float32),
        scratch_types=[
            pltpu.VMEM((R * D,), jnp.float32),
            pltpu.VMEM((CH,), jnp.int32),
            pltpu.VMEM((CH,), jnp.int32),
            pltpu.VMEM((CHW,), jnp.float32),
            pltpu.VMEM((CHW,), jnp.float32),
            pltpu.SemaphoreType.DMA,
            pltpu.SemaphoreType.DMA,
            pltpu.SemaphoreType.DMA,
            pltpu.SemaphoreType.DMA,
        ],
        compiler_params=pltpu.CompilerParams(needs_layout_passes=False),
    )
    def k(tbl_hbm, x_hbm, out_hbm, tbl_v, xb0, xb1, rb0, rb1,
          sx0, sx1, so0, so1):
        wid = lax.axis_index("s") * NC + lax.axis_index("c")
        base = wid * PER_W
        lane = lax.iota(jnp.int32, L)
        lane32 = lane * D
        xbufs, rbufs = (xb0, xb1), (rb0, rb1)
        sxs, sos = (sx0, sx1), (so0, so1)

        pltpu.sync_copy(tbl_hbm, tbl_v)

        def load_x(g):
            return pltpu.async_copy(
                x_hbm.at[pl.ds(base + g * CH, CH)], xbufs[g % 2], sxs[g % 2])

        def gather_chunk(gc, xbuf, rows):
            def group(g2, _):
                pos = lax.rem(gc * CH + g2 * L + lane, 10)
                cw = (xbuf[pl.ds(g2 * L, L)] * 10 + pos) * D
                for l0 in range(0, L, 8):
                    vals = []
                    for l in range(l0, l0 + 8):
                        c = cw[l]
                        vals.append([tbl_v[pl.ds(c + h, L)]
                                     for h in range(0, D, L)])
                    for l in range(l0, l0 + 8):
                        n = g2 * L + l
                        for i, h in enumerate(range(0, D, L)):
                            rows[pl.ds(n * D + h, L)] = vals[l - l0][i]
                return 0

            lax.fori_loop(0, NG, group, 0)

        x_pend = load_x(0)
        out_pend = [None, None]
        for gc in range(NCHUNK):
            b = gc % 2
            nxt = load_x(gc + 1) if gc + 1 < NCHUNK else None
            x_pend.wait()
            x_pend = nxt
            if out_pend[b] is not None:
                out_pend[b].wait()
            gather_chunk(gc, xbufs[b], rbufs[b])
            out_pend[b] = pltpu.async_copy(
                rbufs[b], out_hbm.at[pl.ds((base + gc * CH) * D, CHW)],
                sos[b])
        for h in out_pend:
            if h is not None:
                h.wait()

    return k(tbl, x_flat)


def kernel(x, embed_weight, pe):
    tbl = (embed_weight[:, None, :] + pe[None, :, :]).reshape(R * D)
    x_flat = x.reshape(N).astype(jnp.int32)
    out = _sc_gather(tbl, x_flat)
    return out.reshape(B, P, D)


# speedup vs baseline: 11.3750x; 2.3625x over previous
import functools

import jax
import jax.numpy as jnp
from jax import lax
from jax.experimental import pallas as pl
from jax.experimental.pallas import tpu as pltpu
from jax.experimental.pallas import tpu_sc as plsc

B = 16384
P = 10
D = 32
R = 14 * P
K = P * D
NC, NS = 2, 16
NW = NC * NS
BSL = B // NW
L = 16
BG = BSL // L


def _sc_gather(tbl_t, x_t):
    mesh = plsc.VectorSubcoreMesh(core_axis_name="c", subcore_axis_name="s")

    @functools.partial(
        pl.kernel,
        mesh=mesh,
        out_type=jax.ShapeDtypeStruct((K, B), jnp.float32),
        scratch_types=[
            pltpu.VMEM((D * R,), jnp.float32),
            pltpu.VMEM((BSL,), jnp.int32),
            pltpu.VMEM((BSL,), jnp.int32),
            pltpu.VMEM((D, BSL), jnp.float32),
            pltpu.VMEM((D, BSL), jnp.float32),
            pltpu.SemaphoreType.DMA,
            pltpu.SemaphoreType.DMA,
            pltpu.SemaphoreType.DMA,
            pltpu.SemaphoreType.DMA,
        ],
        compiler_params=pltpu.CompilerParams(needs_layout_passes=False),
    )
    def k(tbl_hbm, x_hbm, out_hbm, tbl_v, xb0, xb1, rb0, rb1,
          sx0, sx1, so0, so1):
        wid = lax.axis_index("s") * NC + lax.axis_index("c")
        b0w = wid * BSL
        xbufs, rbufs = (xb0, xb1), (rb0, rb1)
        sxs, sos = (sx0, sx1), (so0, so1)

        pltpu.sync_copy(tbl_hbm, tbl_v)

        def load_x(p):
            return pltpu.async_copy(
                x_hbm.at[p, pl.ds(b0w, BSL)], xbufs[p % 2], sxs[p % 2])

        x_pend = load_x(0)
        out_pend = [None, None]
        for p in range(P):
            bp = p % 2
            nxt = load_x(p + 1) if p + 1 < P else None
            x_pend.wait()
            x_pend = nxt
            if out_pend[bp] is not None:
                out_pend[bp].wait()
            xbuf, rows = xbufs[bp], rbufs[bp]

            def group(g, _):
                base16 = xbuf[pl.ds(g * L, L)] * 10 + p
                for j in range(D):
                    val = plsc.load_gather(tbl_v, [base16 + j * R])
                    rows[j, pl.ds(g * L, L)] = val
                return 0

            lax.fori_loop(0, BG, group, 0)
            out_pend[bp] = pltpu.async_copy(
                rows, out_hbm.at[pl.ds(p * D, D), pl.ds(b0w, BSL)], sos[bp])
        for h in out_pend:
            if h is not None:
                h.wait()

    return k(tbl_t, x_t)


def kernel(x, embed_weight, pe):
    tbl3 = embed_weight[:, None, :] + pe[None, :, :]
    tbl_t = tbl3.transpose(2, 0, 1).reshape(D * R)
    x_t = x.T.astype(jnp.int32)
    out2 = _sc_gather(tbl_t, x_t)
    return out2.reshape(P, D, B).transpose(2, 0, 1)


# speedup vs baseline: 23.6545x; 2.0795x over previous
import functools

import jax
import jax.numpy as jnp
from jax import lax
from jax.experimental import pallas as pl
from jax.experimental.pallas import tpu as pltpu
from jax.experimental.pallas import tpu_sc as plsc

B = 16384
P = 10
D = 32
R = 14 * P
K = P * D
NC, NS = 2, 16
NW = NC * NS
BSL = B // NW
L = 16
BG = BSL // L


def _sc_gather(tbl_t, x_t):
    mesh = plsc.VectorSubcoreMesh(core_axis_name="c", subcore_axis_name="s")

    @functools.partial(
        pl.kernel,
        mesh=mesh,
        out_type=jax.ShapeDtypeStruct((K, B), jnp.float32),
        scratch_types=[
            pltpu.VMEM((D * R,), jnp.float32),
            pltpu.VMEM((BSL,), jnp.int32),
            pltpu.VMEM((BSL,), jnp.int32),
            pltpu.VMEM((D, BSL), jnp.float32),
            pltpu.VMEM((D, BSL), jnp.float32),
            pltpu.SemaphoreType.DMA,
            pltpu.SemaphoreType.DMA,
            pltpu.SemaphoreType.DMA,
            pltpu.SemaphoreType.DMA,
        ],
        compiler_params=pltpu.CompilerParams(needs_layout_passes=False),
    )
    def k(tbl_hbm, x_hbm, out_hbm, tbl_v, xb0, xb1, rb0, rb1,
          sx0, sx1, so0, so1):
        wid = lax.axis_index("s") * NC + lax.axis_index("c")
        b0w = wid * BSL
        xbufs, rbufs = (xb0, xb1), (rb0, rb1)
        sxs, sos = (sx0, sx1), (so0, so1)

        pltpu.sync_copy(tbl_hbm, tbl_v)

        def load_x(p):
            return pltpu.async_copy(
                x_hbm.at[p, pl.ds(b0w, BSL)], xbufs[p % 2], sxs[p % 2])

        x_pend = load_x(0)
        out_pend = [None, None]
        for p in range(P):
            bp = p % 2
            nxt = load_x(p + 1) if p + 1 < P else None
            x_pend.wait()
            x_pend = nxt
            if out_pend[bp] is not None:
                out_pend[bp].wait()
            xbuf, rows = xbufs[bp], rbufs[bp]

            def load_grp(g):
                base16 = xbuf[pl.ds(g * L, L)] * 10 + p
                return [plsc.load_gather(tbl_v, [base16 + j * R])
                        for j in range(D)]

            def group(g, vals):
                base16 = xbuf[pl.ds(g * L, L)] * 10 + p
                new = []
                for j in range(D):
                    new.append(plsc.load_gather(tbl_v, [base16 + j * R]))
                    rows[j, pl.ds((g - 1) * L, L)] = vals[j]
                return tuple(new)

            vals_last = lax.fori_loop(1, BG, group, tuple(load_grp(0)))
            for j in range(D):
                rows[j, pl.ds((BG - 1) * L, L)] = vals_last[j]
            out_pend[bp] = pltpu.async_copy(
                rows, out_hbm.at[pl.ds(p * D, D), pl.ds(b0w, BSL)], sos[bp])
        for h in out_pend:
            if h is not None:
                h.wait()

    return k(tbl_t, x_t)


def kernel(x, embed_weight, pe):
    tbl3 = embed_weight[:, None, :] + pe[None, :, :]
    tbl_t = tbl3.transpose(2, 0, 1).reshape(D * R)
    x_t = x.T.astype(jnp.int32)
    out2 = _sc_gather(tbl_t, x_t)
    return out2.reshape(P, D, B).transpose(2, 0, 1)
